# Initial kernel scaffold; baseline (speedup 1.0000x reference)
import jax
import jax.numpy as jnp
from jax.experimental import pallas as pl


def kernel(h_L, mask_indices, unmasked_indices, range_r, Wr, br, W1, b1, W2, b2):
    raise NotImplementedError("write your pallas kernel here")


# speedup vs baseline: 2.6192x; 2.6192x over previous
import math

import jax
import jax.numpy as jnp
from jax.experimental import pallas as pl
from jax.experimental.pallas import tpu as pltpu

_F32 = jnp.float32


def _gelu_exact(x):
    z = x * (1.0 / math.sqrt(2.0))
    s = jnp.where(z >= 0.0, 1.0, -1.0)
    a = jnp.abs(z)
    t = 1.0 / (1.0 + 0.3275911 * a)
    p = ((((1.061405429 * t - 1.453152027) * t + 1.421413741) * t
          - 0.284496736) * t + 0.254829592) * t
    erf = s * (1.0 - p * jnp.exp(-a * a))
    return x * 0.5 * (1.0 + erf)


def _stage_a_kernel(u_sm, m_sm, rr_sm,
                    hL, u_row, m_col, w1a, w1b, wr, br, b1f, b2, emat,
                    pre1, pre2, cwT, wexp, wb2, hasf,
                    anch, hmsk):
    b = pl.program_id(0)
    NU = anch.shape[0]
    NM = hmsk.shape[0]
    D = anch.shape[1]

    def gather_anchor(i, carry):
        anch[pl.ds(i, 1), :] = hL[0, pl.ds(u_sm[b, i], 1), :]
        return carry

    jax.lax.fori_loop(0, NU, gather_anchor, 0)

    def gather_mask(i, carry):
        hmsk[pl.ds(i, 1), :] = hL[0, pl.ds(m_sm[b, i], 1), :]
        return carry

    jax.lax.fori_loop(0, NM, gather_mask, 0)

    A = anch[...]
    Hm = hmsk[...]

    pre1[0] = jnp.dot(A, w1a[...], preferred_element_type=_F32)
    pre2[0] = jnp.dot(Hm, w1b[...], preferred_element_type=_F32) + b1f[...]

    logits = jnp.dot(Hm, wr[...], preferred_element_type=_F32) + br[...]
    mxl = jnp.max(logits, axis=1, keepdims=True)
    el = jnp.exp(logits - mxl)
    wgt = el / jnp.sum(el, axis=1, keepdims=True)
    wexp[0] = jnp.dot(wgt, emat[...], preferred_element_type=_F32)
    wb2[0] = jnp.dot(wgt, b2[...], preferred_element_type=_F32)

    rr = rr_sm[0]

    scoresT = jax.lax.dot_general(
        A, Hm, (((1,), (1,)), ((), ())),
        preferred_element_type=_F32) * (1.0 / math.sqrt(D))
    diffT = jnp.abs(u_row[0].reshape(NU, 1) - m_col[0].reshape(1, NM))
    validT = (diffT > 0) & (diffT <= rr)
    smT = jnp.where(validT, scoresT, -1e30)
    mxT = jnp.max(smT, axis=0, keepdims=True)
    eT = jnp.where(validT, jnp.exp(smT - mxT), 0.0)
    denT = jnp.sum(eT, axis=0, keepdims=True)
    cwT[0] = eT / jnp.maximum(denT, 1e-30)

    diffM = jnp.abs(m_col[0].reshape(NM, 1) - u_row[0].reshape(1, NU))
    validM = (diffM > 0) & (diffM <= rr)
    hasf[0] = jnp.max(validM.astype(_F32), axis=1, keepdims=True)


def _stage_b_kernel(m_sm,
                    pre1, pre2, cwT, wexp, wb2, hasf, w2s,
                    delta,
                    wg_s, val_s):
    b = pl.program_id(0)
    NU, NM = cwT.shape[1], cwT.shape[2]
    KDH = pre1.shape[2]
    L, D = delta.shape[1], delta.shape[2]
    CH = 256

    iota_m = jax.lax.broadcasted_iota(jnp.int32, (NM, 1), 0)
    cwT_val = cwT[0]

    def token(m, carry):
        onehot = (iota_m == m).astype(_F32)
        cwcol = jnp.dot(cwT_val, onehot,
                        preferred_element_type=_F32)
        p2row = pre2[0, pl.ds(m, 1), :]
        g = jnp.zeros((1, KDH), _F32)
        for c in range(NU // CH):
            blk = pre1[0, c * CH:(c + 1) * CH, :] + p2row
            act = _gelu_exact(blk)
            g = g + jnp.sum(act * cwcol[c * CH:(c + 1) * CH, :],
                            axis=0, keepdims=True)
        wg_s[pl.ds(m, 1), :] = g * wexp[0, pl.ds(m, 1), :]
        return carry

    jax.lax.fori_loop(0, NM, token, 0)

    val = jnp.dot(wg_s[...], w2s[...], preferred_element_type=_F32)
    val_s[...] = (val + wb2[0]) * hasf[0]

    delta[0] = jnp.zeros((L, D), _F32)

    def scatter(m, carry):
        a = m_sm[b, m]
        delta[0, pl.ds(a, 1), :] = val_s[pl.ds(m, 1), :]
        return carry

    jax.lax.fori_loop(0, NM, scatter, 0)


def kernel(h_L, mask_indices, unmasked_indices, range_r, Wr, br, W1, b1, W2, b2):
    B, L, D = h_L.shape
    NM = mask_indices.shape[1]
    NU = unmasked_indices.shape[1]
    K = Wr.shape[1]
    DH = W1.shape[2]
    KDH = K * DH

    h_L = h_L.astype(_F32)
    u_i = unmasked_indices.astype(jnp.int32)
    m_i = mask_indices.astype(jnp.int32)
    rr = jnp.asarray(range_r, jnp.int32).reshape(1)

    W1a = W1[:, :D, :].transpose(1, 0, 2).reshape(D, KDH)
    W1b = W1[:, D:, :].transpose(1, 0, 2).reshape(D, KDH)
    b1f = b1.reshape(1, KDH)
    W2s = W2.reshape(KDH, D)
    brf = br.reshape(1, K)
    emat = jnp.repeat(jnp.eye(K, dtype=_F32), DH, axis=1)
    u_row = u_i.reshape(B, 1, NU)
    m_col = m_i.reshape(B, NM, 1)

    grid_a = pltpu.PrefetchScalarGridSpec(
        num_scalar_prefetch=3,
        grid=(B,),
        in_specs=[
            pl.BlockSpec((1, L, D), lambda b, *_: (b, 0, 0)),
            pl.BlockSpec((1, 1, NU), lambda b, *_: (b, 0, 0)),
            pl.BlockSpec((1, NM, 1), lambda b, *_: (b, 0, 0)),
            pl.BlockSpec((D, KDH), lambda b, *_: (0, 0)),
            pl.BlockSpec((D, KDH), lambda b, *_: (0, 0)),
            pl.BlockSpec((D, K), lambda b, *_: (0, 0)),
            pl.BlockSpec((1, K), lambda b, *_: (0, 0)),
            pl.BlockSpec((1, KDH), lambda b, *_: (0, 0)),
            pl.BlockSpec((K, D), lambda b, *_: (0, 0)),
            pl.BlockSpec((K, KDH), lambda b, *_: (0, 0)),
        ],
        out_specs=[
            pl.BlockSpec((1, NU, KDH), lambda b, *_: (b, 0, 0)),
            pl.BlockSpec((1, NM, KDH), lambda b, *_: (b, 0, 0)),
            pl.BlockSpec((1, NU, NM), lambda b, *_: (b, 0, 0)),
            pl.BlockSpec((1, NM, KDH), lambda b, *_: (b, 0, 0)),
            pl.BlockSpec((1, NM, D), lambda b, *_: (b, 0, 0)),
            pl.BlockSpec((1, NM, 1), lambda b, *_: (b, 0, 0)),
        ],
        scratch_shapes=[
            pltpu.VMEM((NU, D), _F32),
            pltpu.VMEM((NM, D), _F32),
        ],
    )
    pre1, pre2, cwT, wexp, wb2, hasf = pl.pallas_call(
        _stage_a_kernel,
        grid_spec=grid_a,
        out_shape=[
            jax.ShapeDtypeStruct((B, NU, KDH), _F32),
            jax.ShapeDtypeStruct((B, NM, KDH), _F32),
            jax.ShapeDtypeStruct((B, NU, NM), _F32),
            jax.ShapeDtypeStruct((B, NM, KDH), _F32),
            jax.ShapeDtypeStruct((B, NM, D), _F32),
            jax.ShapeDtypeStruct((B, NM, 1), _F32),
        ],
        compiler_params=pltpu.CompilerParams(
            dimension_semantics=("arbitrary",),
            vmem_limit_bytes=63 * 1024 * 1024),
    )(u_i, m_i, rr,
      h_L, u_row, m_col, W1a, W1b, Wr, brf, b1f, b2, emat)

    grid_b = pltpu.PrefetchScalarGridSpec(
        num_scalar_prefetch=1,
        grid=(B,),
        in_specs=[
            pl.BlockSpec((1, NU, KDH), lambda b, *_: (b, 0, 0)),
            pl.BlockSpec((1, NM, KDH), lambda b, *_: (b, 0, 0)),
            pl.BlockSpec((1, NU, NM), lambda b, *_: (b, 0, 0)),
            pl.BlockSpec((1, NM, KDH), lambda b, *_: (b, 0, 0)),
            pl.BlockSpec((1, NM, D), lambda b, *_: (b, 0, 0)),
            pl.BlockSpec((1, NM, 1), lambda b, *_: (b, 0, 0)),
            pl.BlockSpec((KDH, D), lambda b, *_: (0, 0)),
        ],
        out_specs=pl.BlockSpec((1, L, D), lambda b, *_: (b, 0, 0)),
        scratch_shapes=[
            pltpu.VMEM((NM, KDH), _F32),
            pltpu.VMEM((NM, D), _F32),
        ],
    )
    delta = pl.pallas_call(
        _stage_b_kernel,
        grid_spec=grid_b,
        out_shape=jax.ShapeDtypeStruct((B, L, D), _F32),
        compiler_params=pltpu.CompilerParams(
            dimension_semantics=("arbitrary",),
            vmem_limit_bytes=63 * 1024 * 1024),
    )(m_i, pre1, pre2, cwT, wexp, wb2, hasf, W2s)

    return delta
